# baseline (speedup 1.0000x reference)
import functools

import jax
import jax.numpy as jnp
from jax import lax
from jax.experimental import pallas as pl
from jax.experimental.pallas import tpu as pltpu
from jax.experimental.pallas import tpu_sc as plsc

V = 63
FEAT = 256
D = 64
B = 4096
L = 200
BT = B * L

NC = 2
NS = 16
NW = NC * NS
PER_W = BT // NW
RPC = 256
CHW = RPC * D
GPC = RPC // 16
N_CHUNKS = PER_W // RPC
NBUF = 4


def _table_body(sym_ref, w_ref, b_ref, out_ref):
    out_ref[...] = (
        jnp.dot(sym_ref[...], w_ref[...], preferred_element_type=jnp.float32)
        + b_ref[...]
    )


def _make_table(symbols, W, b):
    sym_pad = jnp.pad(symbols, ((0, 64 - V), (0, 0)))
    return pl.pallas_call(
        _table_body,
        out_shape=jax.ShapeDtypeStruct((64, D), jnp.float32),
    )(sym_pad, W, b.reshape(1, D))


def _sc_gather_body(
    table_hbm, idx_hbm, out_hbm, table_v, idx_v, rb0, rb1, rb2, rb3, *ws
):
    rbufs = (rb0, rb1, rb2, rb3)
    wid = lax.axis_index("s") * NC + lax.axis_index("c")
    pltpu.sync_copy(table_hbm, table_v)
    pltpu.sync_copy(idx_hbm.at[pl.ds(wid * PER_W, PER_W)], idx_v)
    iota64 = lax.iota(jnp.int32, 16) * D

    def write_start(ci, b):
        pltpu.make_async_copy(
            rbufs[b],
            out_hbm.at[pl.ds((wid * N_CHUNKS + ci) * CHW, CHW)],
            ws[b],
        ).start()

    def write_wait(b):
        pltpu.make_async_copy(
            rbufs[b], out_hbm.at[pl.ds(0, CHW)], ws[b]
        ).wait()

    def chunk_step(i, carry):
        for bslot in range(NBUF):
            ci = i * NBUF + bslot

            @pl.when(i >= 1)
            def _():
                write_wait(bslot)

            def group(g, carry2):
                idxv = idx_v[pl.ds(ci * RPC + g * 16, 16)]
                gpos = idxv * D
                spos = iota64 + g * (16 * D)
                for c in range(D):
                    v = plsc.load_gather(table_v, [gpos + c])
                    plsc.store_scatter(rbufs[bslot], [spos + c], v)
                return carry2

            lax.fori_loop(0, GPC, group, 0)
            write_start(ci, bslot)
        return carry

    lax.fori_loop(0, N_CHUNKS // NBUF, chunk_step, 0)
    for b in range(NBUF):
        write_wait(b)


@functools.partial(jax.jit)
def kernel(QR, symbols, W, b):
    table = _make_table(symbols, W, b).reshape(-1)
    idx = QR.reshape(BT).astype(jnp.int32)
    mesh = plsc.VectorSubcoreMesh(core_axis_name="c", subcore_axis_name="s")
    gather = pl.kernel(
        _sc_gather_body,
        out_type=jax.ShapeDtypeStruct((BT * D,), jnp.float32),
        mesh=mesh,
        scratch_types=(
            [
                pltpu.VMEM((64 * D,), jnp.float32),
                pltpu.VMEM((PER_W,), jnp.int32),
                pltpu.VMEM((CHW,), jnp.float32),
                pltpu.VMEM((CHW,), jnp.float32),
                pltpu.VMEM((CHW,), jnp.float32),
                pltpu.VMEM((CHW,), jnp.float32),
            ]
            + [pltpu.SemaphoreType.DMA] * NBUF
        ),
        compiler_params=pltpu.CompilerParams(needs_layout_passes=False),
    )
    out = gather(table, idx)
    return out.reshape(B, L, D)
